# Initial kernel scaffold; baseline (speedup 1.0000x reference)
import functools

import jax
import jax.numpy as jnp
from jax import lax
from jax.experimental import pallas as pl
from jax.experimental.pallas import tpu as pltpu
from jax.experimental.pallas import tpu_sc as plsc

N = 10000
E = 320000
D = 128

NC = 2
NS = 16
NW = NC * NS
EW = E // NW
C = 80
CH = EW // C
RPS = N // NS


def _sc_body(nfeat_hbm, src_hbm, dst_hbm, w_hbm, zeros_hbm, out_hbm,
             src_v, dst_v, w_v, rows_v, hn_sh, sem):
    c = lax.axis_index("c")
    s = lax.axis_index("s")
    wid = c * NS + s

    pltpu.sync_copy(zeros_hbm.at[pl.ds(s * RPS, RPS)],
                    hn_sh.at[pl.ds(s * RPS, RPS)])

    pltpu.sync_copy(src_hbm.at[wid], src_v)
    pltpu.sync_copy(dst_hbm.at[wid], dst_v)
    pltpu.sync_copy(w_hbm.at[wid], w_v)

    plsc.subcore_barrier()

    def chunk(i, carry):
        pltpu.async_copy(nfeat_hbm.at[src_v.at[i]], rows_v, sem).wait()

        def edge(e, carry2):
            wv = jnp.full((16,), w_v[i, e], jnp.float32)
            for j in range(D // 16):
                sl = pl.ds(j * 16, 16)
                rows_v[e, sl] = rows_v[e, sl] * wv
            return carry2

        lax.fori_loop(0, C, edge, 0, unroll=2)

        pltpu.sync_copy(rows_v, hn_sh.at[dst_v.at[i]], add=True)
        return carry

    lax.fori_loop(0, CH, chunk, 0)

    plsc.subcore_barrier()

    pltpu.sync_copy(hn_sh.at[pl.ds(s * RPS, RPS)],
                    out_hbm.at[c, pl.ds(s * RPS, RPS)])


_sc_call = functools.partial(
    pl.kernel,
    out_type=jax.ShapeDtypeStruct((NC, N, D), jnp.float32),
    mesh=plsc.VectorSubcoreMesh(core_axis_name="c", subcore_axis_name="s"),
    scratch_types=[
        pltpu.VMEM((CH, C), jnp.int32),
        pltpu.VMEM((CH, C), jnp.int32),
        pltpu.VMEM((CH, C), jnp.float32),
        pltpu.VMEM((C, D), jnp.float32),
        pltpu.VMEM_SHARED((N, D), jnp.float32),
        pltpu.SemaphoreType.DMA,
    ],
)(_sc_body)


def _tc_body(h_ref, p_ref, w1_ref, w2_ref, hn_ref, out_ref):
    h = h_ref[...]
    hn = p_ref[0] + p_ref[1]
    hn_ref[...] = hn
    a = lax.dot_general(h + hn, w1_ref[...], (((1,), (1,)), ((), ())),
                        precision=lax.Precision.HIGHEST,
                        preferred_element_type=jnp.float32)
    b = lax.dot_general(h * hn, w2_ref[...], (((1,), (1,)), ((), ())),
                        precision=lax.Precision.HIGHEST,
                        preferred_element_type=jnp.float32)
    out_ref[...] = (jnp.where(a > 0, a, 0.01 * a)
                    + jnp.where(b > 0, b, 0.01 * b))


_TB = 1000

_tc_call = pl.pallas_call(
    _tc_body,
    grid=(N // _TB,),
    in_specs=[
        pl.BlockSpec((_TB, D), lambda i: (i, 0)),
        pl.BlockSpec((NC, _TB, D), lambda i: (0, i, 0)),
        pl.BlockSpec((D, D), lambda i: (0, 0)),
        pl.BlockSpec((D, D), lambda i: (0, 0)),
    ],
    out_specs=[
        pl.BlockSpec((_TB, D), lambda i: (i, 0)),
        pl.BlockSpec((_TB, D), lambda i: (i, 0)),
    ],
    out_shape=[
        jax.ShapeDtypeStruct((N, D), jnp.float32),
        jax.ShapeDtypeStruct((N, D), jnp.float32),
    ],
)


def kernel(nfeat, edge_index, edge_weight, W1, W2):
    src = edge_index[0].astype(jnp.int32).reshape(NW, CH, C)
    dst = edge_index[1].astype(jnp.int32).reshape(NW, CH, C)
    w = edge_weight.astype(jnp.float32).reshape(NW, CH, C)
    zeros = jnp.zeros((N, D), jnp.float32)
    partials = _sc_call(nfeat, src, dst, w, zeros)
    hn, out = _tc_call(nfeat, partials, W1, W2)
    return (hn, out)


# speedup vs baseline: 6.3912x; 6.3912x over previous
import functools

import jax
import jax.numpy as jnp
from jax import lax
from jax.experimental import pallas as pl
from jax.experimental.pallas import tpu as pltpu
from jax.experimental.pallas import tpu_sc as plsc

N = 10000
E = 320000
D = 128

NC = 2
NS = 16
NW = NC * NS
EW = E // NW
C = 80
CH = EW // C
SCH = 25
SS = CH // SCH
NP = 10240
RPS = NP // NS


def _sc_body(nfeat_hbm, src_hbm, dst_hbm, w_hbm, zeros_hbm, out_hbm,
             src_v, dst_v, w_v, rows_v, hn_sh, sem):
    c = lax.axis_index("c")
    s = lax.axis_index("s")
    wid = c * NS + s

    pltpu.sync_copy(zeros_hbm.at[pl.ds(s * RPS, RPS)],
                    hn_sh.at[pl.ds(s * RPS, RPS)])

    plsc.subcore_barrier()

    def sup(ss, carry0):
        pltpu.sync_copy(src_hbm.at[wid, ss], src_v)
        pltpu.sync_copy(dst_hbm.at[wid, ss], dst_v)
        pltpu.sync_copy(w_hbm.at[wid, ss], w_v)

        def chunk(i, carry):
            pltpu.async_copy(nfeat_hbm.at[src_v.at[i]], rows_v, sem).wait()

            def group(g, carry2):
                w16 = w_v[i, pl.ds(g * 16, 16)]
                for e16 in range(16):
                    wv = jnp.full((16,), w16[e16], jnp.float32)
                    e = g * 16 + e16
                    for j in range(D // 16):
                        sl = pl.ds(j * 16, 16)
                        rows_v[e, sl] = rows_v[e, sl] * wv
                return carry2

            lax.fori_loop(0, C // 16, group, 0)

            pltpu.sync_copy(rows_v, hn_sh.at[dst_v.at[i]], add=True)
            return carry

        lax.fori_loop(0, SCH, chunk, 0)
        return carry0

    lax.fori_loop(0, SS, sup, 0)

    plsc.subcore_barrier()

    pltpu.sync_copy(hn_sh.at[pl.ds(s * RPS, RPS)],
                    out_hbm.at[c, pl.ds(s * RPS, RPS)])


_sc_call = functools.partial(
    pl.kernel,
    out_type=jax.ShapeDtypeStruct((NC, NP, D), jnp.float32),
    mesh=plsc.VectorSubcoreMesh(core_axis_name="c", subcore_axis_name="s"),
    scratch_types=[
        pltpu.VMEM((SCH, C), jnp.int32),
        pltpu.VMEM((SCH, C), jnp.int32),
        pltpu.VMEM((SCH, C), jnp.float32),
        pltpu.VMEM((C, D), jnp.float32),
        pltpu.VMEM_SHARED((NP, D), jnp.float32),
        pltpu.SemaphoreType.DMA,
    ],
)(_sc_body)


def _tc_body(h_ref, p_ref, w1_ref, w2_ref, hn_ref, out_ref):
    h = h_ref[...]
    hn = p_ref[0] + p_ref[1]
    hn_ref[...] = hn
    a = lax.dot_general(h + hn, w1_ref[...], (((1,), (1,)), ((), ())),
                        precision=lax.Precision.HIGHEST,
                        preferred_element_type=jnp.float32)
    b = lax.dot_general(h * hn, w2_ref[...], (((1,), (1,)), ((), ())),
                        precision=lax.Precision.HIGHEST,
                        preferred_element_type=jnp.float32)
    out_ref[...] = (jnp.where(a > 0, a, 0.01 * a)
                    + jnp.where(b > 0, b, 0.01 * b))


_TB = 1024

_tc_call = pl.pallas_call(
    _tc_body,
    grid=(pl.cdiv(N, _TB),),
    in_specs=[
        pl.BlockSpec((_TB, D), lambda i: (i, 0)),
        pl.BlockSpec((NC, _TB, D), lambda i: (0, i, 0)),
        pl.BlockSpec((D, D), lambda i: (0, 0)),
        pl.BlockSpec((D, D), lambda i: (0, 0)),
    ],
    out_specs=[
        pl.BlockSpec((_TB, D), lambda i: (i, 0)),
        pl.BlockSpec((_TB, D), lambda i: (i, 0)),
    ],
    out_shape=[
        jax.ShapeDtypeStruct((N, D), jnp.float32),
        jax.ShapeDtypeStruct((N, D), jnp.float32),
    ],
)


def kernel(nfeat, edge_index, edge_weight, W1, W2):
    src = edge_index[0].astype(jnp.int32).reshape(NW, SS, SCH, C)
    dst = edge_index[1].astype(jnp.int32).reshape(NW, SS, SCH, C)
    w = edge_weight.astype(jnp.float32).reshape(NW, SS, SCH, C)
    zeros = jnp.zeros((NP, D), jnp.float32)
    partials = _sc_call(nfeat, src, dst, w, zeros)
    hn, out = _tc_call(nfeat, partials, W1, W2)
    return (hn, out)
